# baseline (speedup 1.0000x reference)
import jax
import jax.numpy as jnp
from jax import lax
from jax.experimental import pallas as pl
from jax.experimental.pallas import tpu as pltpu
from jax.experimental.pallas import tpu_sc as plsc

N = 10000
E = 160000
D = 256

NUM_CORES = 2
NUM_SUBCORES = 16
NW = NUM_CORES * NUM_SUBCORES
HALF_N = N // NUM_CORES
EPS = E // NUM_SUBCORES
PIECE = 2000
CHUNK = 48
KEPT_CAP = 10240
ACC_ROWS = 5120
DUMMY_LOCAL = HALF_N
DRAIN_ROWS = 312
DRAIN_TAIL = HALF_N - NUM_SUBCORES * DRAIN_ROWS
ZROWS = ACC_ROWS // NUM_SUBCORES


def _artanh(v):
    v = jnp.clip(v, -1.0 + 1e-5, 1.0 - 1e-5)
    return 0.5 * (jnp.log1p(v) - jnp.log1p(-v))


def _pre_body(x_ref, w_ref, b_ref, h_ref):
    x = x_ref[...]
    nrm = jnp.maximum(jnp.sqrt(jnp.sum(x * x, axis=1, keepdims=True)), 1e-15)
    h = x * (_artanh(nrm) / nrm)
    hw = lax.dot_general(h, w_ref[...], (((1,), (0,)), ((), ())),
                         preferred_element_type=jnp.float32)
    hw = hw + b_ref[...]
    blk = hw.shape[0]
    u = lax.bitcast_convert_type(hw, jnp.int32)
    r = u + 0x7FFF + ((u >> 16) & 1)
    bb = ((r >> 16) & 0xFFFF).reshape(blk, 8, 2, 16)
    word = (bb[:, :, 1, :] << 16) | bb[:, :, 0, :]
    h_ref[...] = word.reshape(blk, 128)


def _post_body(a_ref, o_ref):
    blk = a_ref.shape[0]
    a = jnp.maximum(a_ref[...].reshape(blk, D), 0.0)
    nrm = jnp.maximum(jnp.sqrt(jnp.sum(a * a, axis=1, keepdims=True)), 1e-15)
    o = jnp.tanh(nrm) * a / nrm
    o_ref[...] = jnp.maximum(o, 0.0)


def _part_body(pidx_hbm, kept_hbm, cnt_hbm, piece_v, kept_v, cnt_v, sem):
    c = lax.axis_index("c")
    s = lax.axis_index("s")
    w = c * NUM_SUBCORES + s

    lo = c * (HALF_N << 16)
    hi = lo + (HALF_N << 16)

    def scan_piece(p, n):
        pltpu.sync_copy(pidx_hbm.at[pl.ds(s * EPS + p * PIECE, PIECE)], piece_v)

        def scan_group(g, n):
            wd = piece_v[pl.ds(g * 16, 16)]
            m = (wd >= lo) & (wd < hi)
            plsc.store_compressed(kept_v.at[pl.ds(n, 16)], wd, mask=m)
            return n + jnp.max(plsc.all_reduce_population_count(m))

        return lax.fori_loop(0, PIECE // 16, scan_group, n)

    n = lax.fori_loop(0, EPS // PIECE, scan_piece, jnp.int32(0))

    dummy_w = jnp.zeros((16,), jnp.int32) + ((c * HALF_N + DUMMY_LOCAL) << 16)

    @pl.loop(0, 2 * CHUNK, step=16)
    def _(k):
        kept_v[pl.ds(n + k, 16)] = dummy_w

    cnt_v[...] = jnp.zeros((16,), jnp.int32) + n
    pltpu.sync_copy(kept_v, kept_hbm.at[pl.ds(w * KEPT_CAP, KEPT_CAP)])
    pltpu.sync_copy(cnt_v, cnt_hbm.at[pl.ds(w * 16, 16)])


def _agg_body(h_hbm, kept_hbm, cnt_hbm, out_hbm,
              kept_v, cnt_v, src_a, src_b, dst_a, dst_b,
              gbuf_a, gbuf_b, fbuf_a, fbuf_b,
              acc_sh, gsem_a, gsem_b, ssem_a, ssem_b):
    c = lax.axis_index("c")
    s = lax.axis_index("s")
    w = c * NUM_SUBCORES + s

    @pl.loop(0, CHUNK)
    def _(r):
        @pl.loop(0, D // 2, step=16)
        def _(col):
            fbuf_a[r, 0, pl.ds(col, 16)] = jnp.zeros((16,), jnp.float32)
            fbuf_a[r, 1, pl.ds(col, 16)] = jnp.zeros((16,), jnp.float32)

    @pl.loop(0, ZROWS // CHUNK)
    def _(k):
        pltpu.sync_copy(fbuf_a, acc_sh.at[pl.ds(s * ZROWS + k * CHUNK, CHUNK)])

    ztail = ZROWS - (ZROWS // CHUNK) * CHUNK
    pltpu.sync_copy(
        fbuf_a.at[pl.ds(0, ztail)],
        acc_sh.at[pl.ds(s * ZROWS + (ZROWS // CHUNK) * CHUNK, ztail)])

    plsc.subcore_barrier()

    pltpu.sync_copy(kept_hbm.at[pl.ds(w * KEPT_CAP, KEPT_CAP)], kept_v)
    pltpu.sync_copy(cnt_hbm.at[pl.ds(w * 16, 16)], cnt_v)
    n = cnt_v[...][0]
    ncs = 2 * jnp.maximum((n + 2 * CHUNK - 1) // (2 * CHUNK), 1)

    base = c * HALF_N

    def split_src(t, src_st):
        @pl.loop(0, CHUNK, step=16)
        def _(k):
            v = kept_v[pl.ds(t * CHUNK + k, 16)]
            src_st[pl.ds(k, 16)] = v & 0xFFFF

    def split_dst(t, dst_st):
        @pl.loop(0, CHUNK, step=16)
        def _(k):
            v = kept_v[pl.ds(t * CHUNK + k, 16)]
            dst_st[pl.ds(k, 16)] = (v >> 16) - base

    def fire_gather(src_st, gbuf, gsem):
        pltpu.async_copy(h_hbm.at[src_st], gbuf, gsem)

    def wait_gather(gbuf, gsem):
        pltpu.make_async_copy(h_hbm.at[src_a], gbuf, gsem).wait()

    def convert(gbuf, fbuf):
        @pl.loop(0, CHUNK)
        def _(r):
            for k in range(8):
                wv = gbuf[r, pl.ds(16 * k, 16)]
                ev = plsc.bitcast(wv << 16, jnp.float32)
                od = plsc.bitcast(wv & jnp.int32(-65536), jnp.float32)
                fbuf[r, k // 4, pl.ds(32 * (k % 4), 16)] = ev
                fbuf[r, k // 4, pl.ds(32 * (k % 4) + 16, 16)] = od

    def fire_scat(fbuf, dst_st, ssem):
        pltpu.async_copy(fbuf, acc_sh.at[dst_st], ssem, add=True)

    def wait_scat(fbuf, ssem):
        pltpu.make_async_copy(fbuf, acc_sh.at[dst_a], ssem).wait()

    lanes = ((src_a, dst_a, gbuf_a, fbuf_a, gsem_a, ssem_a),
             (src_b, dst_b, gbuf_b, fbuf_b, gsem_b, ssem_b))

    for k, (src_st, dst_st, gbuf, fbuf, gsem, ssem) in enumerate(lanes):
        split_src(k, src_st)
        fire_gather(src_st, gbuf, gsem)

    @pl.loop(0, ncs, step=2)
    def _(j):
        for k, (src_st, dst_st, gbuf, fbuf, gsem, ssem) in enumerate(lanes):
            t = j + k
            wait_gather(gbuf, gsem)

            @pl.when(t >= 2)
            def _(fbuf=fbuf, ssem=ssem):
                wait_scat(fbuf, ssem)

            split_dst(t, dst_st)
            convert(gbuf, fbuf)
            fire_scat(fbuf, dst_st, ssem)

            @pl.when(t + 2 < ncs)
            def _(t=t, src_st=src_st, gbuf=gbuf, gsem=gsem):
                split_src(t + 2, src_st)
                fire_gather(src_st, gbuf, gsem)

    for k, (src_st, dst_st, gbuf, fbuf, gsem, ssem) in enumerate(lanes):
        wait_scat(fbuf, ssem)

    plsc.subcore_barrier()

    pltpu.sync_copy(acc_sh.at[pl.ds(s * DRAIN_ROWS, DRAIN_ROWS)],
                    out_hbm.at[pl.ds(c * HALF_N + s * DRAIN_ROWS, DRAIN_ROWS)])

    @pl.when(s == 0)
    def _():
        tb = NUM_SUBCORES * DRAIN_ROWS
        pltpu.sync_copy(acc_sh.at[pl.ds(tb, DRAIN_TAIL)],
                        out_hbm.at[pl.ds(c * HALF_N + tb, DRAIN_TAIL)])


@jax.jit
def kernel(x, edge_index, W, b):
    blk = 1000
    grid = N // blk
    h = pl.pallas_call(
        _pre_body,
        grid=(grid,),
        in_specs=[
            pl.BlockSpec((blk, D), lambda i: (i, 0)),
            pl.BlockSpec((D, D), lambda i: (0, 0)),
            pl.BlockSpec((1, D), lambda i: (0, 0)),
        ],
        out_specs=pl.BlockSpec((blk, D // 2), lambda i: (i, 0)),
        out_shape=jax.ShapeDtypeStruct((N, D // 2), jnp.int32),
    )(x, W, b.reshape(1, D))

    packed = (edge_index[1] << 16) | edge_index[0]

    mesh = plsc.VectorSubcoreMesh(core_axis_name="c", subcore_axis_name="s")
    part = pl.kernel(
        _part_body,
        out_type=[
            jax.ShapeDtypeStruct((NW * KEPT_CAP,), jnp.int32),
            jax.ShapeDtypeStruct((NW * 16,), jnp.int32),
        ],
        mesh=mesh,
        scratch_types=[
            pltpu.VMEM((PIECE,), jnp.int32),
            pltpu.VMEM((KEPT_CAP,), jnp.int32),
            pltpu.VMEM((16,), jnp.int32),
            pltpu.SemaphoreType.DMA,
        ],
        compiler_params=pltpu.CompilerParams(needs_layout_passes=False),
    )
    kept, cnt = part(packed)

    agg = pl.kernel(
        _agg_body,
        out_type=jax.ShapeDtypeStruct((N, 2, D // 2), jnp.float32),
        mesh=mesh,
        scratch_types=[
            pltpu.VMEM((KEPT_CAP,), jnp.int32),
            pltpu.VMEM((16,), jnp.int32),
            pltpu.VMEM((CHUNK,), jnp.int32),
            pltpu.VMEM((CHUNK,), jnp.int32),
            pltpu.VMEM((CHUNK,), jnp.int32),
            pltpu.VMEM((CHUNK,), jnp.int32),
            pltpu.VMEM((CHUNK, D // 2), jnp.int32),
            pltpu.VMEM((CHUNK, D // 2), jnp.int32),
            pltpu.VMEM((CHUNK, 2, D // 2), jnp.float32),
            pltpu.VMEM((CHUNK, 2, D // 2), jnp.float32),
            pltpu.VMEM_SHARED((ACC_ROWS, 2, D // 2), jnp.float32),
            pltpu.SemaphoreType.DMA,
            pltpu.SemaphoreType.DMA,
            pltpu.SemaphoreType.DMA,
            pltpu.SemaphoreType.DMA,
        ],
        compiler_params=pltpu.CompilerParams(needs_layout_passes=False),
    )(h, kept, cnt)

    out = pl.pallas_call(
        _post_body,
        grid=(grid,),
        in_specs=[pl.BlockSpec((blk, 2, D // 2), lambda i: (i, 0, 0))],
        out_specs=pl.BlockSpec((blk, D), lambda i: (i, 0)),
        out_shape=jax.ShapeDtypeStruct((N, D), jnp.float32),
    )(agg)
    return out


# speedup vs baseline: 1.7133x; 1.7133x over previous
import jax
import jax.numpy as jnp
from jax import lax
from jax.experimental import pallas as pl
from jax.experimental.pallas import tpu as pltpu
from jax.experimental.pallas import tpu_sc as plsc

N = 10000
E = 160000
D = 256

NUM_CORES = 2
NUM_SUBCORES = 16
NW = NUM_CORES * NUM_SUBCORES
HALF_N = N // NUM_CORES
EPS = E // NUM_SUBCORES
PIECE = 2000
CHUNK = 64
KEPT_CAP = 10240
ACC_ROWS = 5120
DUMMY_LOCAL = HALF_N
DRAIN_ROWS = 312
DRAIN_TAIL = HALF_N - NUM_SUBCORES * DRAIN_ROWS
ZROWS = ACC_ROWS // NUM_SUBCORES


def _artanh(v):
    v = jnp.clip(v, -1.0 + 1e-5, 1.0 - 1e-5)
    return 0.5 * (jnp.log1p(v) - jnp.log1p(-v))


def _pre_body(x_ref, w_ref, b_ref, h_ref):
    x = x_ref[...]
    nrm = jnp.maximum(jnp.sqrt(jnp.sum(x * x, axis=1, keepdims=True)), 1e-15)
    h = x * (_artanh(nrm) / nrm)
    hw = lax.dot_general(h, w_ref[...], (((1,), (0,)), ((), ())),
                         preferred_element_type=jnp.float32)
    hw = hw + b_ref[...]
    h_ref[...] = hw.reshape(hw.shape[0], 2, D // 2)


def _post_body(a_ref, o_ref):
    a = a_ref[...]
    a = jnp.maximum(a.reshape(a.shape[0], D), 0.0)
    nrm = jnp.maximum(jnp.sqrt(jnp.sum(a * a, axis=1, keepdims=True)), 1e-15)
    o = jnp.tanh(nrm) * a / nrm
    o_ref[...] = jnp.maximum(o, 0.0)


def _part_body(pidx_hbm, kept_hbm, cnt_hbm, piece_v, kept_v, cnt_v, sem):
    c = lax.axis_index("c")
    s = lax.axis_index("s")
    w = c * NUM_SUBCORES + s

    lo = c * (HALF_N << 16)
    hi = lo + (HALF_N << 16)

    def scan_piece(p, n):
        pltpu.sync_copy(pidx_hbm.at[pl.ds(s * EPS + p * PIECE, PIECE)], piece_v)

        def scan_group(g, n):
            wd = piece_v[pl.ds(g * 16, 16)]
            m = (wd >= lo) & (wd < hi)
            plsc.store_compressed(kept_v.at[pl.ds(n, 16)], wd, mask=m)
            return n + jnp.max(plsc.all_reduce_population_count(m))

        return lax.fori_loop(0, PIECE // 16, scan_group, n)

    n = lax.fori_loop(0, EPS // PIECE, scan_piece, jnp.int32(0))

    dummy_w = jnp.zeros((16,), jnp.int32) + ((c * HALF_N + DUMMY_LOCAL) << 16)

    @pl.loop(0, 2 * CHUNK, step=16)
    def _(k):
        kept_v[pl.ds(n + k, 16)] = dummy_w

    cnt_v[...] = jnp.zeros((16,), jnp.int32) + n
    pltpu.sync_copy(kept_v, kept_hbm.at[pl.ds(w * KEPT_CAP, KEPT_CAP)])
    pltpu.sync_copy(cnt_v, cnt_hbm.at[pl.ds(w * 16, 16)])


def _agg_body(h_hbm, kept_hbm, cnt_hbm, out_hbm,
              kept_v, cnt_v, src_a, src_b, dst_a, dst_b, buf_a, buf_b,
              acc_sh, sem_a, sem_b):
    c = lax.axis_index("c")
    s = lax.axis_index("s")
    w = c * NUM_SUBCORES + s

    @pl.loop(0, CHUNK)
    def _(r):
        @pl.loop(0, D // 2, step=16)
        def _(col):
            buf_a[r, 0, pl.ds(col, 16)] = jnp.zeros((16,), jnp.float32)
            buf_a[r, 1, pl.ds(col, 16)] = jnp.zeros((16,), jnp.float32)

    @pl.loop(0, ZROWS // CHUNK)
    def _(k):
        pltpu.sync_copy(buf_a, acc_sh.at[pl.ds(s * ZROWS + k * CHUNK, CHUNK)])

    plsc.subcore_barrier()

    pltpu.sync_copy(kept_hbm.at[pl.ds(w * KEPT_CAP, KEPT_CAP)], kept_v)
    pltpu.sync_copy(cnt_hbm.at[pl.ds(w * 16, 16)], cnt_v)
    n = cnt_v[...][0]
    ncs = 2 * jnp.maximum((n + 2 * CHUNK - 1) // (2 * CHUNK), 1)

    base = c * HALF_N

    def unpack(t, src_st, dst_st):
        @pl.loop(0, CHUNK, step=16)
        def _(k):
            v = kept_v[pl.ds(t * CHUNK + k, 16)]
            src_st[pl.ds(k, 16)] = v & 0xFFFF
            dst_st[pl.ds(k, 16)] = (v >> 16) - base

    def fire(src_st, buf, sem):
        pltpu.async_copy(h_hbm.at[src_st], buf, sem)

    def wait(buf, sem):
        pltpu.make_async_copy(h_hbm.at[src_a], buf, sem).wait()

    def scat(buf, dst_st):
        pltpu.sync_copy(buf, acc_sh.at[dst_st], add=True)

    unpack(0, src_a, dst_a)
    fire(src_a, buf_a, sem_a)

    @pl.loop(0, ncs, step=2)
    def _(j):
        unpack(j + 1, src_b, dst_b)
        fire(src_b, buf_b, sem_b)
        wait(buf_a, sem_a)
        scat(buf_a, dst_a)

        @pl.when(j + 2 < ncs)
        def _():
            unpack(j + 2, src_a, dst_a)
            fire(src_a, buf_a, sem_a)

        wait(buf_b, sem_b)
        scat(buf_b, dst_b)

    plsc.subcore_barrier()

    pltpu.sync_copy(acc_sh.at[pl.ds(s * DRAIN_ROWS, DRAIN_ROWS)],
                    out_hbm.at[pl.ds(c * HALF_N + s * DRAIN_ROWS, DRAIN_ROWS)])

    @pl.when(s == 0)
    def _():
        tb = NUM_SUBCORES * DRAIN_ROWS
        pltpu.sync_copy(acc_sh.at[pl.ds(tb, DRAIN_TAIL)],
                        out_hbm.at[pl.ds(c * HALF_N + tb, DRAIN_TAIL)])


@jax.jit
def kernel(x, edge_index, W, b):
    blk = 1000
    grid = N // blk
    h = pl.pallas_call(
        _pre_body,
        grid=(grid,),
        in_specs=[
            pl.BlockSpec((blk, D), lambda i: (i, 0)),
            pl.BlockSpec((D, D), lambda i: (0, 0)),
            pl.BlockSpec((1, D), lambda i: (0, 0)),
        ],
        out_specs=pl.BlockSpec((blk, 2, D // 2), lambda i: (i, 0, 0)),
        out_shape=jax.ShapeDtypeStruct((N, 2, D // 2), jnp.float32),
    )(x, W, b.reshape(1, D))

    packed = (edge_index[1] << 16) | edge_index[0]

    mesh = plsc.VectorSubcoreMesh(core_axis_name="c", subcore_axis_name="s")
    part = pl.kernel(
        _part_body,
        out_type=[
            jax.ShapeDtypeStruct((NW * KEPT_CAP,), jnp.int32),
            jax.ShapeDtypeStruct((NW * 16,), jnp.int32),
        ],
        mesh=mesh,
        scratch_types=[
            pltpu.VMEM((PIECE,), jnp.int32),
            pltpu.VMEM((KEPT_CAP,), jnp.int32),
            pltpu.VMEM((16,), jnp.int32),
            pltpu.SemaphoreType.DMA,
        ],
        compiler_params=pltpu.CompilerParams(needs_layout_passes=False),
    )
    kept, cnt = part(packed)

    agg = pl.kernel(
        _agg_body,
        out_type=jax.ShapeDtypeStruct((N, 2, D // 2), jnp.float32),
        mesh=mesh,
        scratch_types=[
            pltpu.VMEM((KEPT_CAP,), jnp.int32),
            pltpu.VMEM((16,), jnp.int32),
            pltpu.VMEM((CHUNK,), jnp.int32),
            pltpu.VMEM((CHUNK,), jnp.int32),
            pltpu.VMEM((CHUNK,), jnp.int32),
            pltpu.VMEM((CHUNK,), jnp.int32),
            pltpu.VMEM((CHUNK, 2, D // 2), jnp.float32),
            pltpu.VMEM((CHUNK, 2, D // 2), jnp.float32),
            pltpu.VMEM_SHARED((ACC_ROWS, 2, D // 2), jnp.float32),
            pltpu.SemaphoreType.DMA,
            pltpu.SemaphoreType.DMA,
        ],
    )(h, kept, cnt)

    out = pl.pallas_call(
        _post_body,
        grid=(grid,),
        in_specs=[pl.BlockSpec((blk, 2, D // 2), lambda i: (i, 0, 0))],
        out_specs=pl.BlockSpec((blk, D), lambda i: (i, 0)),
        out_shape=jax.ShapeDtypeStruct((N, D), jnp.float32),
    )(agg)
    return out
